# baseline (speedup 1.0000x reference)
import functools

import jax
import jax.numpy as jnp
from jax import lax
from jax.experimental import pallas as pl
from jax.experimental.pallas import tpu as pltpu
from jax.experimental.pallas import tpu_sc as plsc

BVS = 1000000
MD = BVS - 1
L = 16
NC, NS = 2, 16
NW = NC * NS
SUB = 128


def _sc_body(S, CHUNK, D, tk_hbm, em_hbm, sc_hbm, out_hbm,
             tkbuf, idxv, rows, scv, sem):
    nsub = CHUNK // SUB
    wid = lax.axis_index("s") * NC + lax.axis_index("c")
    base = wid * CHUNK

    pltpu.sync_copy(sc_hbm, scv)
    pltpu.sync_copy(tk_hbm.at[pl.ds(base, CHUNK)], tkbuf.at[pl.ds(8, CHUNK)])

    @pl.when(base != 0)
    def _():
        pltpu.sync_copy(tk_hbm.at[pl.ds(base - 8, 8)], tkbuf.at[pl.ds(0, 8)])

    def hash_body(i, _):
        cur = tkbuf[pl.ds(8 + i * L, L)]
        prev = tkbuf[pl.ds(7 + i * L, L)]
        a = jnp.int32(36313) * cur
        b = jnp.int32(27191) * prev
        r = lax.rem(lax.bitwise_xor(a, b), jnp.int32(MD))
        r = jnp.where(r < 0, r + jnp.int32(MD), r)
        pos = base + i * L + lax.iota(jnp.int32, L)
        r = jnp.where((pos & (S - 1)) == 0, jnp.int32(MD), r)
        idxv[i // (SUB // L), pl.ds((i % (SUB // L)) * L, L)] = r
        return 0

    lax.fori_loop(0, CHUNK // L, hash_body, 0, unroll=2)

    copies = [
        pltpu.async_copy(em_hbm.at[idxv.at[j]], rows.at[pl.ds(j * SUB, SUB)], sem)
        for j in range(nsub)
    ]
    for cp in copies:
        cp.wait()

    scale = scv[...]

    def scale_body(r, _):
        for c in range(D // L):
            rows[r, pl.ds(c * L, L)] = rows[r, pl.ds(c * L, L)] * scale
        return 0

    lax.fori_loop(0, CHUNK, scale_body, 0, unroll=2)

    pltpu.sync_copy(rows, out_hbm.at[pl.ds(base, CHUNK)])


def kernel(tk, em_weight, sc):
    B, S = tk.shape
    V, D = em_weight.shape
    N = B * S
    CHUNK = N // NW

    tk_flat = tk.reshape(N).astype(jnp.int32)
    sc_vec = jnp.broadcast_to(sc.astype(jnp.float32), (L,))

    mesh = plsc.VectorSubcoreMesh(core_axis_name="c", subcore_axis_name="s")
    body = functools.partial(_sc_body, S, CHUNK, D)
    out = pl.kernel(
        body,
        mesh=mesh,
        compiler_params=pltpu.CompilerParams(use_tc_tiling_on_sc=False),
        out_type=jax.ShapeDtypeStruct((N, D), jnp.float32),
        scratch_types=[
            pltpu.VMEM((CHUNK + 8,), jnp.int32),
            pltpu.VMEM((CHUNK // SUB, SUB), jnp.int32),
            pltpu.VMEM((CHUNK, D), jnp.float32),
            pltpu.VMEM((L,), jnp.float32),
            pltpu.SemaphoreType.DMA,
        ],
    )(tk_flat, em_weight, sc_vec)
    return out.reshape(B, S, D)
